# baseline (speedup 1.0000x reference)
import functools
import math

import jax
import jax.numpy as jnp
from jax import lax
from jax.experimental import pallas as pl
from jax.experimental.pallas import tpu as pltpu
from jax.experimental.pallas import tpu_sc as plsc

B = 1024; L = 200; V = 100000; D = 128; K = 80; H = 4; FFN = 256
TAU = 256.0; MAXLEN = 80; NLAYERS = 2; PAD = 0
HD = D // H

NC = 2
NS = 16
NW = NC * NS


def _sc_gather(table, ids2d):
    R = ids2d.shape[0]
    assert R % NW == 0
    cpw = R // NW
    ids3d = ids2d.reshape(NW, cpw, 128)
    NBUF = 5
    assert cpw % NBUF == 0
    mesh = plsc.VectorSubcoreMesh(core_axis_name="c", subcore_axis_name="s")

    @functools.partial(
        pl.kernel,
        out_type=jax.ShapeDtypeStruct((R * 128, D), jnp.float32),
        mesh=mesh,
        scratch_types=[
            pltpu.VMEM((cpw, 128), jnp.int32),
        ] + [pltpu.VMEM((128, D), jnp.float32) for _ in range(NBUF)]
          + [pltpu.SemaphoreType.DMA for _ in range(2 * NBUF)],
    )
    def k(table_hbm, ids_hbm, out_hbm, idx_v, *bufs_sems):
        bufs = bufs_sems[:NBUF]
        gsems = bufs_sems[NBUF:2 * NBUF]
        wsems = bufs_sems[2 * NBUF:]
        wid = lax.axis_index("s") * NC + lax.axis_index("c")
        base = wid * cpw
        pltpu.sync_copy(ids_hbm.at[wid], idx_v)
        for j in range(NBUF):
            pltpu.async_copy(table_hbm.at[idx_v.at[j]], bufs[j], gsems[j])

        def outer(t, _):
            cbase = t * NBUF
            for j in range(NBUF):
                c = cbase + j
                pltpu.make_async_copy(
                    table_hbm.at[idx_v.at[c]], bufs[j], gsems[j]).wait()
                pltpu.async_copy(
                    bufs[j], out_hbm.at[pl.ds((base + c) * 128, 128)],
                    wsems[j])

                @pl.when(c + NBUF < cpw)
                def _():
                    pltpu.make_async_copy(
                        bufs[j],
                        out_hbm.at[pl.ds((base + c) * 128, 128)],
                        wsems[j]).wait()
                    pltpu.async_copy(
                        table_hbm.at[idx_v.at[c + NBUF]], bufs[j], gsems[j])
            return 0

        lax.fori_loop(0, cpw // NBUF, outer, 0)
        for j in range(NBUF):
            c = cpw - NBUF + j
            pltpu.make_async_copy(
                bufs[j], out_hbm.at[pl.ds((base + c) * 128, 128)],
                wsems[j]).wait()

    return k(table, ids3d)


_BB_TOPK = 64


def _topk_body(att_ref, q_ref, ids_ref, vals_ref, sel_ref):
    att = att_ref[...]
    q = q_ref[...]
    ids = ids_ref[...]
    s = jnp.sum(att * q[:, None, :], axis=-1)
    pos = lax.broadcasted_iota(jnp.int32, (1, L), 1).astype(jnp.float32)
    decay = jnp.exp(-(L - 1.0 - pos) / TAU)
    s = s + jnp.log(decay + 1e-8)
    s = jnp.where(ids == PAD, -1e9, s)
    lane = lax.broadcasted_iota(jnp.int32, (1, L), 1)
    work = s
    val_cols = []
    sel_cols = []
    for _ in range(K):
        m = jnp.max(work, axis=-1, keepdims=True)
        at_m = work == m
        jmin = jnp.min(jnp.where(at_m, lane, L), axis=-1, keepdims=True)
        chosen = lane == jmin
        val_cols.append(m)
        sel_cols.append(
            jnp.sum(jnp.where(chosen, ids, 0), axis=-1, keepdims=True))
        work = jnp.where(chosen, -jnp.inf, work)
    vals_ref[...] = jnp.concatenate(val_cols, axis=1)
    sel_ref[...] = jnp.concatenate(sel_cols, axis=1)


def _topk(att3, query_vec, seq_ids):
    bb = _BB_TOPK
    grid = (B // bb,)
    return pl.pallas_call(
        _topk_body,
        grid=grid,
        in_specs=[
            pl.BlockSpec((bb, L, D), lambda i: (i, 0, 0)),
            pl.BlockSpec((bb, D), lambda i: (i, 0)),
            pl.BlockSpec((bb, L), lambda i: (i, 0)),
        ],
        out_specs=[
            pl.BlockSpec((bb, K), lambda i: (i, 0)),
            pl.BlockSpec((bb, K), lambda i: (i, 0)),
        ],
        out_shape=[
            jax.ShapeDtypeStruct((B, K), jnp.float32),
            jax.ShapeDtypeStruct((B, K), jnp.int32),
        ],
    )(att3, query_vec, seq_ids)


_BB_TX = 16


def _rms(x, w, eps=1e-6):
    return w * x * lax.rsqrt(jnp.mean(x * x, axis=-1, keepdims=True) + eps)


def _tx_body(x_ref, vals_ref, *refs):
    layer_refs = refs[:11 * NLAYERS]
    aux_wt_ref, aux_b_ref, u_ref, aux_ref = refs[11 * NLAYERS:]
    bb = _BB_TX
    x = x_ref[...].reshape(bb * K, D)
    dm = lax.broadcasted_iota(jnp.int32, (1, 1, D), 2)
    hmasks = [((dm >= h * HD) & (dm < (h + 1) * HD)).astype(jnp.float32)
              for h in range(H)]
    inv_sqrt_hd = 1.0 / math.sqrt(float(HD))

    for li in range(NLAYERS):
        (wqkv, bqkv, wout, bout, n1w, w1, b1, w2, b2, n2w, pmask) = (
            r[...] for r in layer_refs[11 * li:11 * (li + 1)])
        qkv = jnp.dot(x.astype(jnp.bfloat16), wqkv.astype(jnp.bfloat16),
                      preferred_element_type=jnp.float32) + bqkv
        q3 = qkv[:, :D].reshape(bb, K, D)
        k3 = qkv[:, D:2 * D].reshape(bb, K, D)
        v3 = qkv[:, 2 * D:].reshape(bb, K, D)
        hsum = jnp.zeros((bb, K, D), jnp.float32)
        q3b = q3.astype(jnp.bfloat16)
        for h in range(H):
            km = (k3 * hmasks[h]).astype(jnp.bfloat16)
            s = lax.dot_general(
                q3b, km, (((2,), (2,)), ((0,), (0,))),
                preferred_element_type=jnp.float32) * inv_sqrt_hd
            s = s + pmask[None]
            s = s - jnp.max(s, axis=-1, keepdims=True)
            e = jnp.exp(s)
            a = (e / jnp.sum(e, axis=-1, keepdims=True)).astype(jnp.bfloat16)
            hsum = hsum + lax.dot_general(
                a, (v3 * hmasks[h]).astype(jnp.bfloat16),
                (((2,), (1,)), ((0,), (0,))),
                preferred_element_type=jnp.float32)
        hcat = hsum.reshape(bb * K, D).astype(jnp.bfloat16)
        hcat = jnp.dot(hcat, wout.astype(jnp.bfloat16),
                       preferred_element_type=jnp.float32) + bout
        x = _rms(x + hcat, n1w)
        g = jnp.dot(x.astype(jnp.bfloat16), w1.astype(jnp.bfloat16),
                    preferred_element_type=jnp.float32) + b1
        g = 0.5 * g * (1.0 + lax.erf(g * (1.0 / math.sqrt(2.0))))
        h2 = jnp.dot(g.astype(jnp.bfloat16), w2.astype(jnp.bfloat16),
                     preferred_element_type=jnp.float32) + b2
        x = _rms(x + h2, n2w)

    vals = vals_ref[...]
    w = jax.nn.softmax(vals, axis=1)
    x3 = x.reshape(bb, K, D)
    u = jnp.sum(x3 * w[:, :, None], axis=1)
    u_ref[...] = u
    aux_ref[...] = (jnp.dot(u, aux_wt_ref[...],
                            preferred_element_type=jnp.float32)
                    + aux_b_ref[...])


def _transformer(x, vals, layer_arrays, aux_wt, aux_b2):
    bb = _BB_TX
    grid = (B // bb,)
    w_specs = []
    w_args = []
    for arrs in layer_arrays:
        for a in arrs:
            w_specs.append(pl.BlockSpec(a.shape, lambda i, n=a.ndim: (0,) * n))
            w_args.append(a)
    w_specs.append(pl.BlockSpec(aux_wt.shape, lambda i: (0, 0)))
    w_specs.append(pl.BlockSpec(aux_b2.shape, lambda i: (0, 0)))
    return pl.pallas_call(
        _tx_body,
        grid=grid,
        in_specs=[
            pl.BlockSpec((bb, K, D), lambda i: (i, 0, 0)),
            pl.BlockSpec((bb, K), lambda i: (i, 0)),
        ] + w_specs,
        out_specs=[
            pl.BlockSpec((bb, D), lambda i: (i, 0)),
            pl.BlockSpec((bb, 1), lambda i: (i, 0)),
        ],
        out_shape=[
            jax.ShapeDtypeStruct((B, D), jnp.float32),
            jax.ShapeDtypeStruct((B, 1), jnp.float32),
        ],
    )(x, vals, *w_args, aux_wt, aux_b2)


def _pos_bias(rel_emb):
    i = jnp.arange(K)[:, None]
    j = jnp.arange(K)[None, :]
    d = jnp.clip(j - i, -MAXLEN, MAXLEN) + MAXLEN
    return jnp.mean(rel_emb[d], axis=-1)


def kernel(seq_ids, query_vec, emb_att, emb_rep, layers, aux_w, aux_b):
    seq_ids = seq_ids.astype(jnp.int32)
    ids2d = seq_ids.reshape(B * L // 128, 128)
    att_rows = _sc_gather(emb_att, ids2d)
    att3 = att_rows.reshape(B, L, D)
    vals, sel_ids = _topk(att3, query_vec, seq_ids)
    rep_rows = _sc_gather(emb_rep, sel_ids.reshape(B * K // 128, 128))
    x = rep_rows.reshape(B, K, D)

    layer_arrays = []
    for p in layers:
        layer_arrays.append([
            p['in_proj_w'].T,
            p['in_proj_b'].reshape(1, 3 * D),
            p['out_proj_w'].T,
            p['out_proj_b'].reshape(1, D),
            p['norm1_w'].reshape(1, D),
            p['ffn_w1'].T,
            p['ffn_b1'].reshape(1, FFN),
            p['ffn_w2'].T,
            p['ffn_b2'].reshape(1, D),
            p['norm2_w'].reshape(1, D),
            _pos_bias(p['rel_emb']),
        ])
    u, aux = _transformer(x, vals, layer_arrays, aux_w.T, aux_b.reshape(1, 1))
    return u, aux[:, 0]


# speedup vs baseline: 1.4365x; 1.4365x over previous
import functools
import math

import jax
import jax.numpy as jnp
from jax import lax
from jax.experimental import pallas as pl
from jax.experimental.pallas import tpu as pltpu
from jax.experimental.pallas import tpu_sc as plsc

B = 1024; L = 200; V = 100000; D = 128; K = 80; H = 4; FFN = 256
TAU = 256.0; MAXLEN = 80; NLAYERS = 2; PAD = 0
HD = D // H

NC = 2
NS = 16
NW = NC * NS


def _sc_gather(table, ids2d):
    R = ids2d.shape[0]
    assert R % NW == 0
    cpw = R // NW
    ids3d = ids2d.reshape(NW, cpw, 128)
    NBUF = 5
    assert cpw % NBUF == 0
    mesh = plsc.VectorSubcoreMesh(core_axis_name="c", subcore_axis_name="s")

    @functools.partial(
        pl.kernel,
        out_type=jax.ShapeDtypeStruct((R * 128, D), jnp.float32),
        mesh=mesh,
        scratch_types=[
            pltpu.VMEM((cpw, 128), jnp.int32),
        ] + [pltpu.VMEM((128, D), jnp.float32) for _ in range(NBUF)]
          + [pltpu.SemaphoreType.DMA for _ in range(2 * NBUF)],
    )
    def k(table_hbm, ids_hbm, out_hbm, idx_v, *bufs_sems):
        bufs = bufs_sems[:NBUF]
        gsems = bufs_sems[NBUF:2 * NBUF]
        wsems = bufs_sems[2 * NBUF:]
        wid = lax.axis_index("s") * NC + lax.axis_index("c")
        base = wid * cpw
        pltpu.sync_copy(ids_hbm.at[wid], idx_v)
        for j in range(NBUF):
            pltpu.async_copy(table_hbm.at[idx_v.at[j]], bufs[j], gsems[j])

        def outer(t, _):
            cbase = t * NBUF
            for j in range(NBUF):
                c = cbase + j
                pltpu.make_async_copy(
                    table_hbm.at[idx_v.at[c]], bufs[j], gsems[j]).wait()
                pltpu.async_copy(
                    bufs[j], out_hbm.at[pl.ds((base + c) * 128, 128)],
                    wsems[j])

                @pl.when(c + NBUF < cpw)
                def _():
                    pltpu.make_async_copy(
                        bufs[j],
                        out_hbm.at[pl.ds((base + c) * 128, 128)],
                        wsems[j]).wait()
                    pltpu.async_copy(
                        table_hbm.at[idx_v.at[c + NBUF]], bufs[j], gsems[j])
            return 0

        lax.fori_loop(0, cpw // NBUF, outer, 0)
        for j in range(NBUF):
            c = cpw - NBUF + j
            pltpu.make_async_copy(
                bufs[j], out_hbm.at[pl.ds((base + c) * 128, 128)],
                wsems[j]).wait()

    return k(table, ids3d)


_BB_TOPK = 128


def _topk_body(att_ref, q_ref, ids_ref, vals_ref, sel_ref):
    att = att_ref[...]
    q = q_ref[...]
    ids = ids_ref[...]
    s = jnp.sum(att * q[:, None, :], axis=-1)
    pos = lax.broadcasted_iota(jnp.int32, (1, L), 1).astype(jnp.float32)
    decay = jnp.exp(-(L - 1.0 - pos) / TAU)
    s = s + jnp.log(decay + 1e-8)
    s = jnp.where(ids == PAD, -1e9, s)
    lane = lax.broadcasted_iota(jnp.int32, (1, L), 1)
    work = s
    val_cols = []
    sel_cols = []
    for _ in range(K):
        m = jnp.max(work, axis=-1, keepdims=True)
        at_m = work == m
        jmin = jnp.min(jnp.where(at_m, lane, L), axis=-1, keepdims=True)
        chosen = lane == jmin
        val_cols.append(m)
        sel_cols.append(
            jnp.sum(jnp.where(chosen, ids, 0), axis=-1, keepdims=True))
        work = jnp.where(chosen, -jnp.inf, work)
    vals_ref[...] = jnp.concatenate(val_cols, axis=1)
    sel_ref[...] = jnp.concatenate(sel_cols, axis=1)


def _topk(att3, query_vec, seq_ids):
    bb = _BB_TOPK
    grid = (B // bb,)
    return pl.pallas_call(
        _topk_body,
        grid=grid,
        in_specs=[
            pl.BlockSpec((bb, L, D), lambda i: (i, 0, 0)),
            pl.BlockSpec((bb, D), lambda i: (i, 0)),
            pl.BlockSpec((bb, L), lambda i: (i, 0)),
        ],
        out_specs=[
            pl.BlockSpec((bb, K), lambda i: (i, 0)),
            pl.BlockSpec((bb, K), lambda i: (i, 0)),
        ],
        out_shape=[
            jax.ShapeDtypeStruct((B, K), jnp.float32),
            jax.ShapeDtypeStruct((B, K), jnp.int32),
        ],
    )(att3, query_vec, seq_ids)


_BB_TX = 32


def _rms(x, w, eps=1e-6):
    return w * x * lax.rsqrt(jnp.mean(x * x, axis=-1, keepdims=True) + eps)


def _tx_body(x_ref, vals_ref, *refs):
    layer_refs = refs[:11 * NLAYERS]
    aux_wt_ref, aux_b_ref, u_ref, aux_ref = refs[11 * NLAYERS:]
    bb = _BB_TX
    x = x_ref[...].reshape(bb * K, D)
    dm = lax.broadcasted_iota(jnp.int32, (1, 1, D), 2)
    hmasks = [((dm >= h * HD) & (dm < (h + 1) * HD)).astype(jnp.float32)
              for h in range(H)]
    inv_sqrt_hd = 1.0 / math.sqrt(float(HD))

    for li in range(NLAYERS):
        (wqkv, bqkv, wout, bout, n1w, w1, b1, w2, b2, n2w, pmask) = (
            r[...] for r in layer_refs[11 * li:11 * (li + 1)])
        qkv = jnp.dot(x, wqkv, preferred_element_type=jnp.float32) + bqkv
        q3 = qkv[:, :D].reshape(bb, K, D)
        k3 = qkv[:, D:2 * D].reshape(bb, K, D)
        v3 = qkv[:, 2 * D:].reshape(bb, K, D)
        hsum = jnp.zeros((bb, K, D), jnp.float32)
        q3b = q3.astype(jnp.bfloat16)
        for h in range(H):
            km = (k3 * hmasks[h]).astype(jnp.bfloat16)
            s = lax.dot_general(
                q3b, km, (((2,), (2,)), ((0,), (0,))),
                preferred_element_type=jnp.float32) * inv_sqrt_hd
            s = s + pmask[None]
            e = jnp.exp(s)
            a = (e / jnp.sum(e, axis=-1, keepdims=True)).astype(jnp.bfloat16)
            hsum = hsum + lax.dot_general(
                a, (v3 * hmasks[h]).astype(jnp.bfloat16),
                (((2,), (1,)), ((0,), (0,))),
                preferred_element_type=jnp.float32)
        hcat = hsum.reshape(bb * K, D)
        hcat = jnp.dot(hcat, wout, preferred_element_type=jnp.float32) + bout
        x = _rms(x + hcat, n1w)
        g = jnp.dot(x, w1, preferred_element_type=jnp.float32) + b1
        g = 0.5 * g * (1.0 + lax.erf(g * (1.0 / math.sqrt(2.0))))
        h2 = jnp.dot(g, w2, preferred_element_type=jnp.float32) + b2
        x = _rms(x + h2, n2w)

    vals = vals_ref[...]
    w = jax.nn.softmax(vals, axis=1)
    x3 = x.reshape(bb, K, D)
    u = jnp.sum(x3 * w[:, :, None], axis=1)
    u_ref[...] = u
    aux_ref[...] = (jnp.dot(u, aux_wt_ref[...],
                            preferred_element_type=jnp.float32)
                    + aux_b_ref[...])


def _transformer(x, vals, layer_arrays, aux_wt, aux_b2):
    bb = _BB_TX
    grid = (B // bb,)
    w_specs = []
    w_args = []
    for arrs in layer_arrays:
        for a in arrs:
            w_specs.append(pl.BlockSpec(a.shape, lambda i, n=a.ndim: (0,) * n))
            w_args.append(a)
    w_specs.append(pl.BlockSpec(aux_wt.shape, lambda i: (0, 0)))
    w_specs.append(pl.BlockSpec(aux_b2.shape, lambda i: (0, 0)))
    return pl.pallas_call(
        _tx_body,
        grid=grid,
        in_specs=[
            pl.BlockSpec((bb, K, D), lambda i: (i, 0, 0)),
            pl.BlockSpec((bb, K), lambda i: (i, 0)),
        ] + w_specs,
        out_specs=[
            pl.BlockSpec((bb, D), lambda i: (i, 0)),
            pl.BlockSpec((bb, 1), lambda i: (i, 0)),
        ],
        out_shape=[
            jax.ShapeDtypeStruct((B, D), jnp.float32),
            jax.ShapeDtypeStruct((B, 1), jnp.float32),
        ],
    )(x, vals, *w_args, aux_wt, aux_b2)


def _pos_bias(rel_emb):
    i = jnp.arange(K)[:, None]
    j = jnp.arange(K)[None, :]
    d = jnp.clip(j - i, -MAXLEN, MAXLEN) + MAXLEN
    return jnp.mean(rel_emb[d], axis=-1)


def kernel(seq_ids, query_vec, emb_att, emb_rep, layers, aux_w, aux_b):
    seq_ids = seq_ids.astype(jnp.int32)
    ids2d = seq_ids.reshape(B * L // 128, 128)
    att_rows = _sc_gather(emb_att, ids2d)
    att3 = att_rows.reshape(B, L, D)
    vals, sel_ids = _topk(att3, query_vec, seq_ids)
    rep_rows = _sc_gather(emb_rep, sel_ids.reshape(B * K // 128, 128))
    x = rep_rows.reshape(B, K, D)

    layer_arrays = []
    for p in layers:
        layer_arrays.append([
            p['in_proj_w'].T,
            p['in_proj_b'].reshape(1, 3 * D),
            p['out_proj_w'].T,
            p['out_proj_b'].reshape(1, D),
            p['norm1_w'].reshape(1, D),
            p['ffn_w1'].T,
            p['ffn_b1'].reshape(1, FFN),
            p['ffn_w2'].T,
            p['ffn_b2'].reshape(1, D),
            p['norm2_w'].reshape(1, D),
            _pos_bias(p['rel_emb']),
        ])
    u, aux = _transformer(x, vals, layer_arrays, aux_w.T, aux_b.reshape(1, 1))
    return u, aux[:, 0]


# speedup vs baseline: 1.4404x; 1.0027x over previous
import functools
import math

import jax
import jax.numpy as jnp
from jax import lax
from jax.experimental import pallas as pl
from jax.experimental.pallas import tpu as pltpu
from jax.experimental.pallas import tpu_sc as plsc

B = 1024; L = 200; V = 100000; D = 128; K = 80; H = 4; FFN = 256
TAU = 256.0; MAXLEN = 80; NLAYERS = 2; PAD = 0
HD = D // H

NC = 2
NS = 16
NW = NC * NS


def _sc_gather(table, ids2d):
    R = ids2d.shape[0]
    assert R % NW == 0
    cpw = R // NW
    ids3d = ids2d.reshape(NW, cpw, 128)
    NBUF = 5
    assert cpw % NBUF == 0
    mesh = plsc.VectorSubcoreMesh(core_axis_name="c", subcore_axis_name="s")

    @functools.partial(
        pl.kernel,
        out_type=jax.ShapeDtypeStruct((R * 128, D), jnp.float32),
        mesh=mesh,
        scratch_types=[
            pltpu.VMEM((cpw, 128), jnp.int32),
        ] + [pltpu.VMEM((128, D), jnp.float32) for _ in range(NBUF)]
          + [pltpu.SemaphoreType.DMA for _ in range(2 * NBUF)],
    )
    def k(table_hbm, ids_hbm, out_hbm, idx_v, *bufs_sems):
        bufs = bufs_sems[:NBUF]
        gsems = bufs_sems[NBUF:2 * NBUF]
        wsems = bufs_sems[2 * NBUF:]
        wid = lax.axis_index("s") * NC + lax.axis_index("c")
        base = wid * cpw
        pltpu.sync_copy(ids_hbm.at[wid], idx_v)
        for j in range(NBUF):
            pltpu.async_copy(table_hbm.at[idx_v.at[j]], bufs[j], gsems[j])

        def outer(t, _):
            cbase = t * NBUF
            for j in range(NBUF):
                c = cbase + j
                pltpu.make_async_copy(
                    table_hbm.at[idx_v.at[c]], bufs[j], gsems[j]).wait()
                pltpu.async_copy(
                    bufs[j], out_hbm.at[pl.ds((base + c) * 128, 128)],
                    wsems[j])

                @pl.when(c + NBUF < cpw)
                def _():
                    pltpu.make_async_copy(
                        bufs[j],
                        out_hbm.at[pl.ds((base + c) * 128, 128)],
                        wsems[j]).wait()
                    pltpu.async_copy(
                        table_hbm.at[idx_v.at[c + NBUF]], bufs[j], gsems[j])
            return 0

        lax.fori_loop(0, cpw // NBUF, outer, 0)
        for j in range(NBUF):
            c = cpw - NBUF + j
            pltpu.make_async_copy(
                bufs[j], out_hbm.at[pl.ds((base + c) * 128, 128)],
                wsems[j]).wait()

    return k(table, ids3d)


_BB_TOPK = 128


def _topk_body(att_ref, q_ref, ids_ref, vals_ref, sel_ref):
    att = att_ref[...]
    q = q_ref[...]
    ids = ids_ref[...]
    s = jnp.sum(att * q[:, None, :], axis=-1)
    pos = lax.broadcasted_iota(jnp.int32, (1, L), 1).astype(jnp.float32)
    decay = jnp.exp(-(L - 1.0 - pos) / TAU)
    s = s + jnp.log(decay + 1e-8)
    s = jnp.where(ids == PAD, -1e9, s)
    lane = lax.broadcasted_iota(jnp.int32, (1, L), 1)
    work = s
    val_cols = []
    sel_cols = []
    for _ in range(K):
        m = jnp.max(work, axis=-1, keepdims=True)
        at_m = work == m
        jmin = jnp.min(jnp.where(at_m, lane, L), axis=-1, keepdims=True)
        chosen = lane == jmin
        val_cols.append(m)
        sel_cols.append(
            jnp.sum(jnp.where(chosen, ids, 0), axis=-1, keepdims=True))
        work = jnp.where(chosen, -jnp.inf, work)
    vals_ref[...] = jnp.concatenate(val_cols, axis=1)
    sel_ref[...] = jnp.concatenate(sel_cols, axis=1)


def _topk(att3, query_vec, seq_ids):
    bb = _BB_TOPK
    grid = (B // bb,)
    return pl.pallas_call(
        _topk_body,
        grid=grid,
        in_specs=[
            pl.BlockSpec((bb, L, D), lambda i: (i, 0, 0)),
            pl.BlockSpec((bb, D), lambda i: (i, 0)),
            pl.BlockSpec((bb, L), lambda i: (i, 0)),
        ],
        out_specs=[
            pl.BlockSpec((bb, K), lambda i: (i, 0)),
            pl.BlockSpec((bb, K), lambda i: (i, 0)),
        ],
        out_shape=[
            jax.ShapeDtypeStruct((B, K), jnp.float32),
            jax.ShapeDtypeStruct((B, K), jnp.int32),
        ],
    )(att3, query_vec, seq_ids)


_BB_TX = 64


def _rms(x, w, eps=1e-6):
    return w * x * lax.rsqrt(jnp.mean(x * x, axis=-1, keepdims=True) + eps)


def _tx_body(x_ref, vals_ref, *refs):
    layer_refs = refs[:11 * NLAYERS]
    aux_wt_ref, aux_b_ref, u_ref, aux_ref = refs[11 * NLAYERS:]
    bb = _BB_TX
    x = x_ref[...].reshape(bb * K, D)
    dm = lax.broadcasted_iota(jnp.int32, (1, 1, D), 2)
    hmasks = [((dm >= h * HD) & (dm < (h + 1) * HD)).astype(jnp.float32)
              for h in range(H)]
    inv_sqrt_hd = 1.0 / math.sqrt(float(HD))

    for li in range(NLAYERS):
        (wqkv, bqkv, wout, bout, n1w, w1, b1, w2, b2, n2w, pmask) = (
            r[...] for r in layer_refs[11 * li:11 * (li + 1)])
        qkv = jnp.dot(x, wqkv, preferred_element_type=jnp.float32) + bqkv
        q3 = qkv[:, :D].reshape(bb, K, D)
        k3 = qkv[:, D:2 * D].reshape(bb, K, D)
        v3 = qkv[:, 2 * D:].reshape(bb, K, D)
        hsum = jnp.zeros((bb, K, D), jnp.float32)
        q3b = q3.astype(jnp.bfloat16)
        for h in range(H):
            km = (k3 * hmasks[h]).astype(jnp.bfloat16)
            s = lax.dot_general(
                q3b, km, (((2,), (2,)), ((0,), (0,))),
                preferred_element_type=jnp.float32) * inv_sqrt_hd
            s = s + pmask[None]
            e = jnp.exp(s)
            a = (e / jnp.sum(e, axis=-1, keepdims=True)).astype(jnp.bfloat16)
            hsum = hsum + lax.dot_general(
                a, (v3 * hmasks[h]).astype(jnp.bfloat16),
                (((2,), (1,)), ((0,), (0,))),
                preferred_element_type=jnp.float32)
        hcat = hsum.reshape(bb * K, D)
        hcat = jnp.dot(hcat, wout, preferred_element_type=jnp.float32) + bout
        x = _rms(x + hcat, n1w)
        g = jnp.dot(x, w1, preferred_element_type=jnp.float32) + b1
        g = 0.5 * g * (1.0 + lax.erf(g * (1.0 / math.sqrt(2.0))))
        h2 = jnp.dot(g, w2, preferred_element_type=jnp.float32) + b2
        x = _rms(x + h2, n2w)

    vals = vals_ref[...]
    w = jax.nn.softmax(vals, axis=1)
    x3 = x.reshape(bb, K, D)
    u = jnp.sum(x3 * w[:, :, None], axis=1)
    u_ref[...] = u
    aux_ref[...] = (jnp.dot(u, aux_wt_ref[...],
                            preferred_element_type=jnp.float32)
                    + aux_b_ref[...])


def _transformer(x, vals, layer_arrays, aux_wt, aux_b2):
    bb = _BB_TX
    grid = (B // bb,)
    w_specs = []
    w_args = []
    for arrs in layer_arrays:
        for a in arrs:
            w_specs.append(pl.BlockSpec(a.shape, lambda i, n=a.ndim: (0,) * n))
            w_args.append(a)
    w_specs.append(pl.BlockSpec(aux_wt.shape, lambda i: (0, 0)))
    w_specs.append(pl.BlockSpec(aux_b2.shape, lambda i: (0, 0)))
    return pl.pallas_call(
        _tx_body,
        grid=grid,
        in_specs=[
            pl.BlockSpec((bb, K, D), lambda i: (i, 0, 0)),
            pl.BlockSpec((bb, K), lambda i: (i, 0)),
        ] + w_specs,
        out_specs=[
            pl.BlockSpec((bb, D), lambda i: (i, 0)),
            pl.BlockSpec((bb, 1), lambda i: (i, 0)),
        ],
        out_shape=[
            jax.ShapeDtypeStruct((B, D), jnp.float32),
            jax.ShapeDtypeStruct((B, 1), jnp.float32),
        ],
    )(x, vals, *w_args, aux_wt, aux_b2)


def _pos_bias(rel_emb):
    i = jnp.arange(K)[:, None]
    j = jnp.arange(K)[None, :]
    d = jnp.clip(j - i, -MAXLEN, MAXLEN) + MAXLEN
    return jnp.mean(rel_emb[d], axis=-1)


def kernel(seq_ids, query_vec, emb_att, emb_rep, layers, aux_w, aux_b):
    seq_ids = seq_ids.astype(jnp.int32)
    ids2d = seq_ids.reshape(B * L // 128, 128)
    att_rows = _sc_gather(emb_att, ids2d)
    att3 = att_rows.reshape(B, L, D)
    vals, sel_ids = _topk(att3, query_vec, seq_ids)
    rep_rows = _sc_gather(emb_rep, sel_ids.reshape(B * K // 128, 128))
    x = rep_rows.reshape(B, K, D)

    layer_arrays = []
    for p in layers:
        layer_arrays.append([
            p['in_proj_w'].T,
            p['in_proj_b'].reshape(1, 3 * D),
            p['out_proj_w'].T,
            p['out_proj_b'].reshape(1, D),
            p['norm1_w'].reshape(1, D),
            p['ffn_w1'].T,
            p['ffn_b1'].reshape(1, FFN),
            p['ffn_w2'].T,
            p['ffn_b2'].reshape(1, D),
            p['norm2_w'].reshape(1, D),
            _pos_bias(p['rel_emb']),
        ])
    u, aux = _transformer(x, vals, layer_arrays, aux_w.T, aux_b.reshape(1, 1))
    return u, aux[:, 0]
